# baseline (speedup 1.0000x reference)
import functools

import jax
import jax.numpy as jnp
from jax import lax
from jax.experimental import pallas as pl
from jax.experimental.pallas import tpu as pltpu
from jax.experimental.pallas import tpu_sc as plsc

_N_SRC = 2048
_N_TGT = 768
_D = 1024
_TOP_N_SIM = 5
_RANK_K = 4
_TAU = 0.07
_TOP_RANKED = _N_TGT * 2 // 3
_NEG = -3.0
_BIG_I = 2 ** 30

_NC = 2
_NS = 16
_NW = _NC * _NS
_ROWS_PER_W = _N_TGT // _NW
_PAD_W = 32
_CHUNKS = _N_SRC // 16


def _sim_body(src_ref, lab_ref, tgt_ref, sim_ref, enc_ref):
    src = src_ref[...]
    tgt = tgt_ref[...]
    lab = lab_ref[...]
    sn = jnp.maximum(jnp.sqrt(jnp.sum(src * src, axis=1, keepdims=True)), 1e-12)
    tn = jnp.maximum(jnp.sqrt(jnp.sum(tgt * tgt, axis=1, keepdims=True)), 1e-12)
    srcn = src / sn
    tgtn = tgt / tn
    sim_ref[...] = lax.dot_general(
        tgtn, srcn, (((1,), (1,)), ((), ())),
        preferred_element_type=jnp.float32,
    )
    col1 = lax.broadcasted_iota(jnp.int32, (1, _N_SRC), 1)
    enc_ref[...] = col1 * 128 + lab


def _sc_assign_body(sim_hbm, enc_hbm, asg_hbm, row_v, enc_v, lab5_v, asg_v):
    c = lax.axis_index("c")
    s = lax.axis_index("s")
    wid = s * _NC + c
    base = wid * _ROWS_PER_W

    pltpu.sync_copy(enc_hbm.at[:], enc_v)

    def row_body(r, _):
        pltpu.sync_copy(sim_hbm.at[base + r], row_v)

        excl = []
        ls = []
        for _t in range(_TOP_N_SIM):
            exc = list(excl)

            def scan(j, carry, exc=exc):
                bv, be = carry
                v = row_v[pl.ds(j * 16, 16)]
                ev = enc_v[pl.ds(j * 16, 16)]
                cond = (v > bv) | ((v == bv) & (ev < be))
                for ex in exc:
                    cond = cond & (ev != ex)
                bv = jnp.where(cond, v, bv)
                be = jnp.where(cond, ev, be)
                return (bv, be)

            bv0 = jnp.full((16,), _NEG, jnp.float32)
            be0 = jnp.full((16,), _BIG_I, jnp.int32)
            bv, be = lax.fori_loop(0, _CHUNKS, scan, (bv0, be0))
            rv = bv[0]
            re = be[0]
            for i in range(1, 16):
                c = (bv[i] > rv) | ((bv[i] == rv) & (be[i] < re))
                rv = jnp.where(c, bv[i], rv)
                re = jnp.where(c, be[i], re)
            excl.append(re)
            ls.append(jnp.bitwise_and(re, 127))

        best_key = jnp.int32(-_BIG_I)
        asg = jnp.int32(0)
        for a in range(_TOP_N_SIM):
            cnt = jnp.int32(0)
            for b in range(_TOP_N_SIM):
                cnt = cnt + (ls[a] == ls[b]).astype(jnp.int32)
            key = cnt * 1048576 - ls[a]
            take = key > best_key
            asg = jnp.where(take, ls[a], asg)
            best_key = jnp.maximum(key, best_key)
        lane = lax.iota(jnp.int32, 16) == 0
        old = asg_v[pl.ds(r, 16)]
        asg_v[pl.ds(r, 16)] = jnp.where(lane, jnp.full((16,), asg, jnp.int32), old)
        return 0

    lax.fori_loop(0, _ROWS_PER_W, row_body, 0)
    pltpu.sync_copy(asg_v.at[pl.ds(0, _PAD_W)], asg_hbm.at[pl.ds(wid * _PAD_W, _PAD_W)])


def _sc_assign(sim, enc):
    mesh = plsc.VectorSubcoreMesh(core_axis_name="c", subcore_axis_name="s")
    fn = functools.partial(
        pl.kernel,
        mesh=mesh,
        out_type=jax.ShapeDtypeStruct((_NW * _PAD_W,), jnp.int32),
        scratch_types=[
            pltpu.VMEM((_N_SRC,), jnp.float32),
            pltpu.VMEM((_N_SRC,), jnp.int32),
            pltpu.VMEM((16,), jnp.int32),
            pltpu.VMEM((_PAD_W + 16,), jnp.int32),
        ],
    )(_sc_assign_body)
    padded = fn(sim, enc)
    return padded.reshape(_NW, _PAD_W)[:, :_ROWS_PER_W].reshape(_N_TGT)


def _finish_body(sim_ref, lab_ref, asg_ref, out_ref):
    sim = sim_ref[...]
    lab = lab_ref[...]
    assigned = asg_ref[...]

    like = lab == assigned

    def top_k_sum(mask):
        w = jnp.where(mask, sim, _NEG)
        s = jnp.zeros((_N_TGT, 1), jnp.float32)
        rem = jnp.full((_N_TGT, 1), float(_RANK_K), jnp.float32)
        for _ in range(_RANK_K):
            vmax = jnp.max(w, axis=1, keepdims=True)
            hit = w == vmax
            cduck = jnp.sum(hit.astype(jnp.float32), axis=1, keepdims=True)
            take = jnp.minimum(cduck, rem)
            s = s + jnp.where(vmax > -2.0, vmax * take, 0.0)
            rem = rem - take
            w = jnp.where(hit, _NEG, w)
        return s

    nln_sum = top_k_sum(like)
    nun_sum = top_k_sum(jnp.logical_not(like))
    scores = nln_sum / nun_sum

    m = jnp.max(sim, axis=1, keepdims=True)
    e = jnp.exp((sim - m) * (1.0 / _TAU))
    den = jnp.sum(e, axis=1, keepdims=True)
    num = jnp.sum(jnp.where(like, e, 0.0), axis=1, keepdims=True)
    lg = jnp.log(num / den + 1e-6)

    rI = lax.broadcasted_iota(jnp.int32, (_N_TGT, _N_TGT), 0)
    cI = lax.broadcasted_iota(jnp.int32, (_N_TGT, _N_TGT), 1)
    s_bc = jnp.broadcast_to(scores, (_N_TGT, _N_TGT))
    s_rv = jnp.sum(jnp.where(rI == cI, s_bc, 0.0), axis=0, keepdims=True)
    beats = (s_rv > scores) | ((s_rv == scores) & (cI < rI))
    rank = jnp.sum(beats.astype(jnp.int32), axis=1, keepdims=True)
    selected = rank < _TOP_RANKED

    total = jnp.sum(jnp.where(selected, lg, 0.0), axis=0, keepdims=True)
    out_ref[...] = -total / _TOP_RANKED


def kernel(source_features, source_labels, target_features):
    lab2d = source_labels.reshape(1, _N_SRC).astype(jnp.int32)
    sim, enc2d = pl.pallas_call(
        _sim_body,
        out_shape=(
            jax.ShapeDtypeStruct((_N_TGT, _N_SRC), jnp.float32),
            jax.ShapeDtypeStruct((1, _N_SRC), jnp.int32),
        ),
    )(source_features, lab2d, target_features)
    assigned = _sc_assign(sim, enc2d.reshape(_N_SRC))
    out = pl.pallas_call(
        _finish_body,
        out_shape=jax.ShapeDtypeStruct((1, 1), jnp.float32),
    )(sim, lab2d, assigned.reshape(_N_TGT, 1))
    return out[0, 0]


# speedup vs baseline: 4.5934x; 4.5934x over previous
import jax
import jax.numpy as jnp
from jax import lax
from jax.experimental import pallas as pl

_N_SRC = 2048
_N_TGT = 768
_D = 1024
_TOP_N_SIM = 5
_RANK_K = 4
_TAU = 0.07
_TOP_RANKED = _N_TGT * 2 // 3
_NEG = -3.0
_BIG_I = 2 ** 30


def _msc_body(src_ref, lab_ref, tgt_ref, out_ref):
    src = src_ref[...]
    tgt = tgt_ref[...]
    lab = lab_ref[...]

    sn = jnp.maximum(jnp.sqrt(jnp.sum(src * src, axis=1, keepdims=True)), 1e-12)
    tn = jnp.maximum(jnp.sqrt(jnp.sum(tgt * tgt, axis=1, keepdims=True)), 1e-12)
    srcn = src / sn
    tgtn = tgt / tn

    sim = lax.dot_general(
        tgtn, srcn, (((1,), (1,)), ((), ())),
        preferred_element_type=jnp.float32,
    )

    col = lax.broadcasted_iota(jnp.int32, (_N_TGT, _N_SRC), 1)
    enc = col * 128 + lab

    simw = sim
    top_labs = []
    for _ in range(_TOP_N_SIM):
        vmax = jnp.max(simw, axis=1, keepdims=True)
        kmin = jnp.min(jnp.where(simw == vmax, enc, _BIG_I), axis=1, keepdims=True)
        top_labs.append(jnp.bitwise_and(kmin, 127))
        simw = jnp.where(enc == kmin, _NEG, simw)

    keys = []
    for a in range(_TOP_N_SIM):
        cnt = jnp.zeros((_N_TGT, 1), jnp.int32)
        for b in range(_TOP_N_SIM):
            cnt = cnt + (top_labs[a] == top_labs[b]).astype(jnp.int32)
        keys.append(cnt * 1048576 - top_labs[a])
    assigned = top_labs[0]
    best_key = keys[0]
    for a in range(1, _TOP_N_SIM):
        better = keys[a] > best_key
        assigned = jnp.where(better, top_labs[a], assigned)
        best_key = jnp.maximum(keys[a], best_key)

    like = lab == assigned

    def top_k_sum(mask):
        w = jnp.where(mask, sim, _NEG)
        s = jnp.zeros((_N_TGT, 1), jnp.float32)
        rem = jnp.full((_N_TGT, 1), float(_RANK_K), jnp.float32)
        for _ in range(_RANK_K):
            vmax = jnp.max(w, axis=1, keepdims=True)
            hit = w == vmax
            c = jnp.sum(hit.astype(jnp.float32), axis=1, keepdims=True)
            take = jnp.minimum(c, rem)
            s = s + jnp.where(vmax > -2.0, vmax * take, 0.0)
            rem = rem - take
            w = jnp.where(hit, _NEG, w)
        return s

    nln_sum = top_k_sum(like)
    nun_sum = top_k_sum(jnp.logical_not(like))
    scores = nln_sum / nun_sum

    m = jnp.max(sim, axis=1, keepdims=True)
    e = jnp.exp((sim - m) * (1.0 / _TAU))
    den = jnp.sum(e, axis=1, keepdims=True)
    num = jnp.sum(jnp.where(like, e, 0.0), axis=1, keepdims=True)
    lg = jnp.log(num / den + 1e-6)

    rI = lax.broadcasted_iota(jnp.int32, (_N_TGT, _N_TGT), 0)
    cI = lax.broadcasted_iota(jnp.int32, (_N_TGT, _N_TGT), 1)
    s_bc = jnp.broadcast_to(scores, (_N_TGT, _N_TGT))
    s_rv = jnp.sum(jnp.where(rI == cI, s_bc, 0.0), axis=0, keepdims=True)
    beats = (s_rv > scores) | ((s_rv == scores) & (cI < rI))
    rank = jnp.sum(beats.astype(jnp.int32), axis=1, keepdims=True)
    selected = rank < _TOP_RANKED

    total = jnp.sum(jnp.where(selected, lg, 0.0), axis=0, keepdims=True)
    out_ref[...] = -total / _TOP_RANKED


def kernel(source_features, source_labels, target_features):
    lab2d = source_labels.reshape(1, _N_SRC).astype(jnp.int32)
    out = pl.pallas_call(
        _msc_body,
        out_shape=jax.ShapeDtypeStruct((1, 1), jnp.float32),
    )(source_features, lab2d, target_features)
    return out[0, 0]
